# baseline (speedup 1.0000x reference)
import functools

import jax
import jax.numpy as jnp
from jax import lax
from jax.experimental import pallas as pl
from jax.experimental.pallas import tpu as pltpu
from jax.experimental.pallas import tpu_sc as plsc

_LANE = 128
_G = 4


@functools.lru_cache(maxsize=None)
def _build(n_rows, vocab, dim):
    info = plsc.get_sparse_core_info()
    nc, ns = info.num_cores, info.num_subcores
    nw = nc * ns
    rpw = n_rows // nw
    ng = rpw // _G

    mesh = plsc.VectorSubcoreMesh(core_axis_name="c", subcore_axis_name="s")

    @functools.partial(
        pl.kernel,
        mesh=mesh,
        compiler_params=pltpu.CompilerParams(use_tc_tiling_on_sc=False),
        out_type=jax.ShapeDtypeStruct((n_rows, _LANE, dim), jnp.float32),
        scratch_types=[
            pltpu.VMEM((rpw * _LANE,), jnp.int32),
            pltpu.VMEM((2, _G, _LANE, dim), jnp.float32),
            pltpu.SemaphoreType.DMA,
            pltpu.SemaphoreType.DMA,
        ],
    )
    def emb_gather(table_hbm, idx_hbm, out_hbm, idx_v, rows_v, gsem, wsem):
        wid = lax.axis_index("s") * nc + lax.axis_index("c")
        r0 = wid * rpw
        pltpu.sync_copy(idx_hbm.at[pl.ds(r0 * _LANE, rpw * _LANE)], idx_v)
        wprev = None
        for j in range(ng):
            slot = j & 1
            gds = []
            for b in range(_G):
                gds.append(pltpu.async_copy(
                    table_hbm.at[idx_v.at[pl.ds((j * _G + b) * _LANE, _LANE)]],
                    rows_v.at[slot, b],
                    gsem,
                ))
            for g in gds:
                g.wait()
            if wprev is not None:
                wprev.wait()
            wprev = pltpu.async_copy(
                rows_v.at[slot],
                out_hbm.at[pl.ds(r0 + j * _G, _G)],
                wsem,
            )
        wprev.wait()

    return emb_gather


def kernel(input_variable, emb_weight):
    batch, hist = input_variable.shape
    vocab, dim = emb_weight.shape
    n = batch * hist
    idx_flat = input_variable.reshape(n)
    if idx_flat.dtype != jnp.int32:
        idx_flat = idx_flat.astype(jnp.int32)
    out = _build(n // _LANE, vocab, dim)(emb_weight, idx_flat)
    return out.reshape(batch, hist, dim)
